# initial kernel scaffold (unmeasured)
import jax
import jax.numpy as jnp
from jax import lax
from jax.experimental import pallas as pl
from jax.experimental.pallas import tpu as pltpu

N_DEV = 8


def kernel(A, B):
    m_per, k = A.shape
    _, n = B.shape

    def body(a_ref, b_ref, out_ref, comm_ref, send_sems, recv_sems):
        my = lax.axis_index("i")
        left = lax.rem(my - 1 + N_DEV, N_DEV)
        right = lax.rem(my + 1, N_DEV)

        barrier_sem = pltpu.get_barrier_semaphore()
        for nbr in (left, right):
            pl.semaphore_signal(
                barrier_sem, inc=1,
                device_id=(nbr,), device_id_type=pl.DeviceIdType.MESH,
            )
        pl.semaphore_wait(barrier_sem, 2)

        b_bf = b_ref[:, :].astype(jnp.bfloat16)

        comm_ref[N_DEV - 1, :, :] = a_ref[:, :].astype(jnp.bfloat16)
        out_ref[pl.ds(my * m_per, m_per), :] = jnp.dot(
            comm_ref[N_DEV - 1, :, :], b_bf, preferred_element_type=jnp.float32
        )

        for h in range(N_DEV - 1):
            src_slot = (N_DEV - 1) if h == 0 else (h - 1)
            rdma = pltpu.make_async_remote_copy(
                src_ref=comm_ref.at[src_slot],
                dst_ref=comm_ref.at[h],
                send_sem=send_sems.at[h],
                recv_sem=recv_sems.at[h],
                device_id=(right,),
                device_id_type=pl.DeviceIdType.MESH,
            )
            rdma.start()
            rdma.wait()

            origin = lax.rem(my - h - 1 + N_DEV, N_DEV)
            out_ref[pl.ds(origin * m_per, m_per), :] = jnp.dot(
                comm_ref[h, :, :], b_bf, preferred_element_type=jnp.float32
            )

    return pl.pallas_call(
        body,
        out_shape=jax.ShapeDtypeStruct((N_DEV * m_per, n), jnp.float32),
        in_specs=[
            pl.BlockSpec(memory_space=pltpu.VMEM),
            pl.BlockSpec(memory_space=pltpu.VMEM),
        ],
        out_specs=pl.BlockSpec(memory_space=pltpu.VMEM),
        scratch_shapes=[
            pltpu.VMEM((N_DEV, m_per, k), jnp.bfloat16),
            pltpu.SemaphoreType.DMA((N_DEV - 1,)),
            pltpu.SemaphoreType.DMA((N_DEV - 1,)),
        ],
        compiler_params=pltpu.CompilerParams(collective_id=0),
    )(A, B)


# baseline (device time: 142358 ns/iter reference)
import jax
import jax.numpy as jnp
from jax import lax
from jax.experimental import pallas as pl
from jax.experimental.pallas import tpu as pltpu

N_DEV = 8


def kernel(A, B):
    m_per, k = A.shape
    _, n = B.shape

    def body(a_ref, b_ref, out_ref, comm_ref, send_sems, recv_sems):
        my = lax.axis_index("i")
        left = lax.rem(my - 1 + N_DEV, N_DEV)
        right = lax.rem(my + 1, N_DEV)

        barrier_sem = pltpu.get_barrier_semaphore()
        for nbr in (left, right):
            pl.semaphore_signal(
                barrier_sem, inc=1,
                device_id=(nbr,), device_id_type=pl.DeviceIdType.MESH,
            )
        pl.semaphore_wait(barrier_sem, 2)

        b_bf = b_ref[:, :].astype(jnp.bfloat16)

        comm_ref[N_DEV - 1, :, :] = a_ref[:, :].astype(jnp.bfloat16)
        out_ref[pl.ds(my * m_per, m_per), :] = jnp.dot(
            comm_ref[N_DEV - 1, :, :], b_bf, preferred_element_type=jnp.float32
        )

        for h in range(N_DEV - 1):
            src_slot = (N_DEV - 1) if h == 0 else (h - 1)
            rdma = pltpu.make_async_remote_copy(
                src_ref=comm_ref.at[src_slot],
                dst_ref=comm_ref.at[h],
                send_sem=send_sems.at[h],
                recv_sem=recv_sems.at[h],
                device_id=(right,),
                device_id_type=pl.DeviceIdType.MESH,
            )
            rdma.start()
            rdma.wait()

            origin = lax.rem(my - h - 1 + N_DEV, N_DEV)
            out_ref[pl.ds(origin * m_per, m_per), :] = jnp.dot(
                comm_ref[h, :, :], b_bf, preferred_element_type=jnp.float32
            )

    return pl.pallas_call(
        body,
        out_shape=jax.ShapeDtypeStruct((N_DEV * m_per, n), jnp.float32),
        in_specs=[
            pl.BlockSpec(memory_space=pltpu.VMEM),
            pl.BlockSpec(memory_space=pltpu.VMEM),
        ],
        out_specs=pl.BlockSpec(memory_space=pltpu.VMEM),
        scratch_shapes=[
            pltpu.VMEM((N_DEV, m_per, k), jnp.bfloat16),
            pltpu.SemaphoreType.DMA((N_DEV - 1,)),
            pltpu.SemaphoreType.DMA((N_DEV - 1,)),
        ],
        compiler_params=pltpu.CompilerParams(
            collective_id=0, vmem_limit_bytes=100 * 1024 * 1024
        ),
    )(A, B)


# device time: 96454 ns/iter; 1.4759x vs baseline; 1.4759x over previous
import jax
import jax.numpy as jnp
from jax import lax
from jax.experimental import pallas as pl
from jax.experimental.pallas import tpu as pltpu

N_DEV = 8


def kernel(A, B):
    m_per, k = A.shape
    _, n = B.shape
    half = m_per // 2

    def body(a_ref, b_ref, out_ref, comm_r, comm_l,
             send_r, recv_r, send_l, recv_l):
        my = lax.axis_index("i")
        left = lax.rem(my - 1 + N_DEV, N_DEV)
        right = lax.rem(my + 1, N_DEV)

        barrier_sem = pltpu.get_barrier_semaphore()
        for nbr in (left, right):
            pl.semaphore_signal(
                barrier_sem, inc=1,
                device_id=(nbr,), device_id_type=pl.DeviceIdType.MESH,
            )
        pl.semaphore_wait(barrier_sem, 2)

        b_bf = b_ref[:, :].astype(jnp.bfloat16)

        comm_r[N_DEV - 1, :, :] = a_ref[0:half, :].astype(jnp.bfloat16)
        comm_l[N_DEV - 1, :, :] = a_ref[half:m_per, :].astype(jnp.bfloat16)

        def make(h):
            src = (N_DEV - 1) if h == 0 else (h - 1)
            rr = pltpu.make_async_remote_copy(
                src_ref=comm_r.at[src], dst_ref=comm_r.at[h],
                send_sem=send_r.at[h], recv_sem=recv_r.at[h],
                device_id=(right,), device_id_type=pl.DeviceIdType.MESH,
            )
            rl = pltpu.make_async_remote_copy(
                src_ref=comm_l.at[src], dst_ref=comm_l.at[h],
                send_sem=send_l.at[h], recv_sem=recv_l.at[h],
                device_id=(left,), device_id_type=pl.DeviceIdType.MESH,
            )
            return rr, rl

        cur = make(0)
        cur[0].start()
        cur[1].start()

        out_ref[pl.ds(my * m_per, half), :] = jnp.dot(
            comm_r[N_DEV - 1, :, :], b_bf, preferred_element_type=jnp.float32
        )
        out_ref[pl.ds(my * m_per + half, half), :] = jnp.dot(
            comm_l[N_DEV - 1, :, :], b_bf, preferred_element_type=jnp.float32
        )

        for h in range(N_DEV - 1):
            rr, rl = cur
            rr.wait_recv()
            rl.wait_recv()
            if h < N_DEV - 2:
                cur = make(h + 1)
                cur[0].start()
                cur[1].start()
            origin_r = lax.rem(my - h - 1 + N_DEV, N_DEV)
            origin_l = lax.rem(my + h + 1, N_DEV)
            out_ref[pl.ds(origin_r * m_per, half), :] = jnp.dot(
                comm_r[h, :, :], b_bf, preferred_element_type=jnp.float32
            )
            out_ref[pl.ds(origin_l * m_per + half, half), :] = jnp.dot(
                comm_l[h, :, :], b_bf, preferred_element_type=jnp.float32
            )
            rr.wait_send()
            rl.wait_send()

    return pl.pallas_call(
        body,
        out_shape=jax.ShapeDtypeStruct((N_DEV * m_per, n), jnp.float32),
        in_specs=[
            pl.BlockSpec(memory_space=pltpu.VMEM),
            pl.BlockSpec(memory_space=pltpu.VMEM),
        ],
        out_specs=pl.BlockSpec(memory_space=pltpu.VMEM),
        scratch_shapes=[
            pltpu.VMEM((N_DEV, half, k), jnp.bfloat16),
            pltpu.VMEM((N_DEV, half, k), jnp.bfloat16),
            pltpu.SemaphoreType.DMA((N_DEV - 1,)),
            pltpu.SemaphoreType.DMA((N_DEV - 1,)),
            pltpu.SemaphoreType.DMA((N_DEV - 1,)),
            pltpu.SemaphoreType.DMA((N_DEV - 1,)),
        ],
        compiler_params=pltpu.CompilerParams(
            collective_id=0, vmem_limit_bytes=100 * 1024 * 1024
        ),
    )(A, B)


# device time: 95264 ns/iter; 1.4944x vs baseline; 1.0125x over previous
import jax
import jax.numpy as jnp
from jax import lax
from jax.experimental import pallas as pl
from jax.experimental.pallas import tpu as pltpu

N_DEV = 8


def kernel(A, B):
    m_per, k = A.shape
    _, n = B.shape
    half = m_per // 2

    def body(a_ref, b_ref, out_ref, comm_r, comm_l,
             send_r, recv_r, send_l, recv_l):
        my = lax.axis_index("i")
        left = lax.rem(my - 1 + N_DEV, N_DEV)
        right = lax.rem(my + 1, N_DEV)

        barrier_sem = pltpu.get_barrier_semaphore()
        for nbr in (left, right):
            pl.semaphore_signal(
                barrier_sem, inc=1,
                device_id=(nbr,), device_id_type=pl.DeviceIdType.MESH,
            )
        pl.semaphore_wait(barrier_sem, 2)

        b_bf = b_ref[:, :].astype(jnp.bfloat16)

        comm_r[N_DEV - 1, :, :] = a_ref[0:half, :].astype(jnp.bfloat16)
        comm_l[N_DEV - 1, :, :] = a_ref[half:m_per, :].astype(jnp.bfloat16)

        def make(h):
            src = (N_DEV - 1) if h == 0 else (h - 1)
            rr = pltpu.make_async_remote_copy(
                src_ref=comm_r.at[src], dst_ref=comm_r.at[h],
                send_sem=send_r.at[h], recv_sem=recv_r.at[h],
                device_id=(right,), device_id_type=pl.DeviceIdType.MESH,
            )
            rl = pltpu.make_async_remote_copy(
                src_ref=comm_l.at[src], dst_ref=comm_l.at[h],
                send_sem=send_l.at[h], recv_sem=recv_l.at[h],
                device_id=(left,), device_id_type=pl.DeviceIdType.MESH,
            )
            return rr, rl

        cur = make(0)
        cur[0].start()
        cur[1].start()

        out_ref[pl.ds(my * m_per, half), :] = jnp.dot(
            comm_r[N_DEV - 1, :, :], b_bf, preferred_element_type=jnp.float32
        )
        out_ref[pl.ds(my * m_per + half, half), :] = jnp.dot(
            comm_l[N_DEV - 1, :, :], b_bf, preferred_element_type=jnp.float32
        )

        for h in range(N_DEV - 1):
            rr, rl = cur
            rr.wait_recv()
            rl.wait_recv()
            if h < N_DEV - 2:
                cur = make(h + 1)
                cur[0].start()
                cur[1].start()
            rr.wait_send()
            rl.wait_send()

    return pl.pallas_call(
        body,
        out_shape=jax.ShapeDtypeStruct((N_DEV * m_per, n), jnp.float32),
        in_specs=[
            pl.BlockSpec(memory_space=pltpu.VMEM),
            pl.BlockSpec(memory_space=pltpu.VMEM),
        ],
        out_specs=pl.BlockSpec(memory_space=pltpu.VMEM),
        scratch_shapes=[
            pltpu.VMEM((N_DEV, half, k), jnp.bfloat16),
            pltpu.VMEM((N_DEV, half, k), jnp.bfloat16),
            pltpu.SemaphoreType.DMA((N_DEV - 1,)),
            pltpu.SemaphoreType.DMA((N_DEV - 1,)),
            pltpu.SemaphoreType.DMA((N_DEV - 1,)),
            pltpu.SemaphoreType.DMA((N_DEV - 1,)),
        ],
        compiler_params=pltpu.CompilerParams(
            collective_id=0, vmem_limit_bytes=100 * 1024 * 1024
        ),
    )(A, B)


# device time: 84110 ns/iter; 1.6925x vs baseline; 1.1326x over previous
import jax
import jax.numpy as jnp
from jax import lax
from jax.experimental import pallas as pl
from jax.experimental.pallas import tpu as pltpu

N_DEV = 8
N_SUB = 2


def kernel(A, B):
    m_per, k = A.shape
    _, n = B.shape
    half = m_per // 2
    sub = half // N_SUB

    def body(a_ref, b_ref, out_ref, comm_r, comm_l,
             send_r, recv_r, send_l, recv_l):
        my = lax.axis_index("i")
        left = lax.rem(my - 1 + N_DEV, N_DEV)
        right = lax.rem(my + 1, N_DEV)

        barrier_sem = pltpu.get_barrier_semaphore()
        for nbr in (left, right):
            pl.semaphore_signal(
                barrier_sem, inc=1,
                device_id=(nbr,), device_id_type=pl.DeviceIdType.MESH,
            )
        pl.semaphore_wait(barrier_sem, 2)

        b_bf = b_ref[:, :].astype(jnp.bfloat16)

        comm_r[N_DEV - 1, :, :] = a_ref[0:half, :].astype(jnp.bfloat16)
        comm_l[N_DEV - 1, :, :] = a_ref[half:m_per, :].astype(jnp.bfloat16)

        def make(h, s):
            src = (N_DEV - 1) if h == 0 else (h - 1)
            rows = pl.ds(s * sub, sub)
            rr = pltpu.make_async_remote_copy(
                src_ref=comm_r.at[src, rows, :], dst_ref=comm_r.at[h, rows, :],
                send_sem=send_r.at[h, s], recv_sem=recv_r.at[h, s],
                device_id=(right,), device_id_type=pl.DeviceIdType.MESH,
            )
            rl = pltpu.make_async_remote_copy(
                src_ref=comm_l.at[src, rows, :], dst_ref=comm_l.at[h, rows, :],
                send_sem=send_l.at[h, s], recv_sem=recv_l.at[h, s],
                device_id=(left,), device_id_type=pl.DeviceIdType.MESH,
            )
            return rr, rl

        rdmas = {}
        for s in range(N_SUB):
            rdmas[(0, s)] = make(0, s)
            rdmas[(0, s)][0].start()
            rdmas[(0, s)][1].start()

        out_ref[pl.ds(my * m_per, half), :] = jnp.dot(
            comm_r[N_DEV - 1, :, :], b_bf, preferred_element_type=jnp.float32
        )
        out_ref[pl.ds(my * m_per + half, half), :] = jnp.dot(
            comm_l[N_DEV - 1, :, :], b_bf, preferred_element_type=jnp.float32
        )

        for h in range(N_DEV - 1):
            for s in range(N_SUB):
                rr, rl = rdmas[(h, s)]
                rr.wait_recv()
                rl.wait_recv()
                if h < N_DEV - 2:
                    rdmas[(h + 1, s)] = make(h + 1, s)
                    rdmas[(h + 1, s)][0].start()
                    rdmas[(h + 1, s)][1].start()
            origin_r = lax.rem(my - h - 1 + N_DEV, N_DEV)
            origin_l = lax.rem(my + h + 1, N_DEV)
            out_ref[pl.ds(origin_r * m_per, half), :] = jnp.dot(
                comm_r[h, :, :], b_bf, preferred_element_type=jnp.float32
            )
            out_ref[pl.ds(origin_l * m_per + half, half), :] = jnp.dot(
                comm_l[h, :, :], b_bf, preferred_element_type=jnp.float32
            )
            for s in range(N_SUB):
                rr, rl = rdmas[(h, s)]
                rr.wait_send()
                rl.wait_send()

    return pl.pallas_call(
        body,
        out_shape=jax.ShapeDtypeStruct((N_DEV * m_per, n), jnp.float32),
        in_specs=[
            pl.BlockSpec(memory_space=pltpu.VMEM),
            pl.BlockSpec(memory_space=pltpu.VMEM),
        ],
        out_specs=pl.BlockSpec(memory_space=pltpu.VMEM),
        scratch_shapes=[
            pltpu.VMEM((N_DEV, half, k), jnp.bfloat16),
            pltpu.VMEM((N_DEV, half, k), jnp.bfloat16),
            pltpu.SemaphoreType.DMA((N_DEV - 1, N_SUB)),
            pltpu.SemaphoreType.DMA((N_DEV - 1, N_SUB)),
            pltpu.SemaphoreType.DMA((N_DEV - 1, N_SUB)),
            pltpu.SemaphoreType.DMA((N_DEV - 1, N_SUB)),
        ],
        compiler_params=pltpu.CompilerParams(
            collective_id=0, vmem_limit_bytes=100 * 1024 * 1024
        ),
    )(A, B)
